# baseline (speedup 1.0000x reference)
import functools

import jax
import jax.numpy as jnp
from jax import lax
from jax.experimental import pallas as pl
from jax.experimental.pallas import tpu as pltpu
from jax.experimental.pallas import tpu_sc as plsc

D = 128
K = 20
NC = 2
NS = 16
NW = NC * NS
IDX_MINOR = 128


def _sc_gather_sum(idx4, E, zeros_blk, B, CB, n_sub):
    mesh = plsc.VectorSubcoreMesh(core_axis_name="c", subcore_axis_name="s")

    def body(idx_hbm, e_hbm, z_hbm, out_hbm, idx_v, acc_v, sem):
        w = lax.axis_index("s") * NC + lax.axis_index("c")
        pltpu.sync_copy(idx_hbm.at[w], idx_v)
        pltpu.sync_copy(z_hbm, acc_v)

        @pl.loop(0, n_sub)
        def _chunk(c):
            dst = acc_v.at[pl.ds(c * IDX_MINOR, IDX_MINOR)]
            for k in range(K):
                pltpu.async_copy(e_hbm.at[idx_v.at[k, c]], dst, sem, add=True)
            for k in range(K):
                pltpu.make_async_copy(e_hbm.at[idx_v.at[k, c]], dst, sem).wait()

        pltpu.sync_copy(acc_v, out_hbm.at[pl.ds(w * CB, CB)])

    run = pl.kernel(
        body,
        out_type=jax.ShapeDtypeStruct((B, D), jnp.float32),
        mesh=mesh,
        scratch_types=[
            pltpu.VMEM((K, n_sub, IDX_MINOR), jnp.int32),
            pltpu.VMEM((CB, D), jnp.float32),
            pltpu.SemaphoreType.DMA,
        ],
    )
    return run(idx4, E, zeros_blk)


def _epilogue(S, genes, doses, types2, Wt, b2, E0, B):
    BLK = 2048
    grid = (B // BLK,)

    def body(s_ref, g_ref, d_ref, t_ref, wt_ref, b_ref, e0_ref, o_ref):
        g = g_ref[...]
        nv = jnp.sum((g >= 0).astype(jnp.float32), axis=1, keepdims=True)
        e0 = e0_ref[...]
        s = s_ref[...] - (K - nv) * e0
        pooled = s / jnp.maximum(nv, 1.0)
        wt = wt_ref[...]
        mod = (d_ref[...] * wt[0:1, :]
               + jnp.where(t_ref[...] == 0, wt[1:2, :], wt[2:3, :])
               + b_ref[...])
        o_ref[...] = pooled + mod

    return pl.pallas_call(
        body,
        grid=grid,
        in_specs=[
            pl.BlockSpec((BLK, D), lambda i: (i, 0)),
            pl.BlockSpec((BLK, K), lambda i: (i, 0)),
            pl.BlockSpec((BLK, 1), lambda i: (i, 0)),
            pl.BlockSpec((BLK, 1), lambda i: (i, 0)),
            pl.BlockSpec((3, D), lambda i: (0, 0)),
            pl.BlockSpec((1, D), lambda i: (0, 0)),
            pl.BlockSpec((1, D), lambda i: (0, 0)),
        ],
        out_specs=pl.BlockSpec((BLK, D), lambda i: (i, 0)),
        out_shape=jax.ShapeDtypeStruct((B, D), jnp.float32),
    )(S, genes, doses, types2, Wt, b2, E0)


@functools.partial(jax.jit, static_argnames=())
def kernel(perturbation_genes, doses, types, E, W, b):
    B = perturbation_genes.shape[0]
    CB = B // NW
    n_sub = CB // IDX_MINOR

    safe = jnp.maximum(perturbation_genes, 0).astype(jnp.int32)
    idx4 = (safe.T.reshape(K, NW, n_sub, IDX_MINOR)
            .transpose(1, 0, 2, 3))
    zeros_blk = jnp.zeros((CB, D), jnp.float32)

    S = _sc_gather_sum(idx4, E, zeros_blk, B, CB, n_sub)

    types2 = types.reshape(B, 1).astype(jnp.int32)
    Wt = W.T
    b2 = b.reshape(1, D)
    E0 = lax.slice(E, (0, 0), (1, D))
    return _epilogue(S, perturbation_genes, doses, types2, Wt, b2, E0, B)


# speedup vs baseline: 3.3876x; 3.3876x over previous
import functools

import jax
import jax.numpy as jnp
from jax import lax
from jax.experimental import pallas as pl
from jax.experimental.pallas import tpu as pltpu
from jax.experimental.pallas import tpu_sc as plsc

D = 128
K = 20
NC = 2
NS = 16
NW = NC * NS
LANES = 16
CHUNK = 128
R = 5000
SENT = -1


def _sc_gather_sum(idx2, E, zeros_blk, B, CB):
    vocab = E.shape[0]
    n_phases = vocab // R
    n_sub = CB // CHUNK
    n_entries = K * CB
    mesh = plsc.VectorSubcoreMesh(core_axis_name="c", subcore_axis_name="s")

    def body(idx_hbm, e_hbm, z_hbm, out_hbm, idx_v, phase_v, acc_v,
             shard_v, sem):
        w = lax.axis_index("s") * NC + lax.axis_index("c")
        s = lax.axis_index("s")
        pltpu.sync_copy(idx_hbm.at[w], idx_v)
        pltpu.sync_copy(z_hbm, acc_v)

        @pl.loop(0, n_phases)
        def _phase(p):
            lo = p * R

            @pl.when(s == 0)
            def _stage():
                pltpu.sync_copy(e_hbm.at[pl.ds(lo, R)], shard_v)

            @pl.loop(0, n_entries // LANES, unroll=8)
            def _rewrite(i):
                raw = idx_v[pl.ds(i * LANES, LANES)]
                inr = (raw >= lo) & (raw < lo + R)
                phase_v[pl.ds(i * LANES, LANES)] = jnp.where(
                    inr, raw - lo, SENT)

            plsc.subcore_barrier()

            @pl.loop(0, n_sub)
            def _chunk(c):
                dst = acc_v.at[pl.ds(c * CHUNK, CHUNK)]
                cps = []
                for k in range(K):
                    src = shard_v.at[plsc.Indices(
                        phase_v.at[pl.ds(k * CB + c * CHUNK, CHUNK)],
                        ignored_value=SENT)]
                    cps.append(pltpu.async_copy(src, dst, sem, add=True))
                for cp in cps:
                    cp.wait()

            plsc.subcore_barrier()

        pltpu.sync_copy(acc_v, out_hbm.at[pl.ds(w * CB, CB)])

    run = pl.kernel(
        body,
        out_type=jax.ShapeDtypeStruct((B, D), jnp.float32),
        mesh=mesh,
        scratch_types=[
            pltpu.VMEM((n_entries,), jnp.int32),
            pltpu.VMEM((n_entries,), jnp.int32),
            pltpu.VMEM((CB, D), jnp.float32),
            pltpu.VMEM_SHARED((R, D), jnp.float32),
            pltpu.SemaphoreType.DMA,
        ],
    )
    return run(idx2, E, zeros_blk)


def _epilogue(S, genes, doses, types2, Wt, b2, B):
    BLK = 2048
    grid = (B // BLK,)

    def body(s_ref, g_ref, d_ref, t_ref, wt_ref, b_ref, o_ref):
        g = g_ref[...]
        nv = jnp.sum((g >= 0).astype(jnp.float32), axis=1, keepdims=True)
        pooled = s_ref[...] / jnp.maximum(nv, 1.0)
        wt = wt_ref[...]
        mod = (d_ref[...] * wt[0:1, :]
               + jnp.where(t_ref[...] == 0, wt[1:2, :], wt[2:3, :])
               + b_ref[...])
        o_ref[...] = pooled + mod

    return pl.pallas_call(
        body,
        grid=grid,
        in_specs=[
            pl.BlockSpec((BLK, D), lambda i: (i, 0)),
            pl.BlockSpec((BLK, K), lambda i: (i, 0)),
            pl.BlockSpec((BLK, 1), lambda i: (i, 0)),
            pl.BlockSpec((BLK, 1), lambda i: (i, 0)),
            pl.BlockSpec((3, D), lambda i: (0, 0)),
            pl.BlockSpec((1, D), lambda i: (0, 0)),
        ],
        out_specs=pl.BlockSpec((BLK, D), lambda i: (i, 0)),
        out_shape=jax.ShapeDtypeStruct((B, D), jnp.float32),
    )(S, genes, doses, types2, Wt, b2)


def kernel(perturbation_genes, doses, types, E, W, b):
    B = perturbation_genes.shape[0]
    CB = B // NW

    raw = perturbation_genes.astype(jnp.int32)
    idx2 = raw.T.reshape(K, NW, CB).transpose(1, 0, 2).reshape(NW, K * CB)
    zeros_blk = jnp.zeros((CB, D), jnp.float32)

    S = _sc_gather_sum(idx2, E, zeros_blk, B, CB)

    types2 = types.reshape(B, 1).astype(jnp.int32)
    Wt = W.T
    b2 = b.reshape(1, D)
    return _epilogue(S, perturbation_genes, doses, types2, Wt, b2, B)


# speedup vs baseline: 5.5131x; 1.6274x over previous
import functools

import jax
import jax.numpy as jnp
from jax import lax
from jax.experimental import pallas as pl
from jax.experimental.pallas import tpu as pltpu
from jax.experimental.pallas import tpu_sc as plsc

D = 128
K = 20
NC = 2
NS = 16
NW = NC * NS
LANES = 16
CHUNK = 128
R = 10000
SENT = -1
PASSES = 2


def _sc_gather_sum(idx3, E, zeros_blk, B, CB):
    vocab = E.shape[0]
    n_phases = vocab // R
    n_sub = CB // CHUNK
    n_entries = K * CB
    half = B // PASSES
    mesh = plsc.VectorSubcoreMesh(core_axis_name="c", subcore_axis_name="s")

    def body(idx_hbm, e_hbm, z_hbm, out_hbm, idx_v, phase_v, acc_v,
             shard_v, sem):
        w = lax.axis_index("s") * NC + lax.axis_index("c")
        s = lax.axis_index("s")

        for pi in range(PASSES):
            pltpu.sync_copy(idx_hbm.at[pi, w], idx_v)
            pltpu.sync_copy(z_hbm, acc_v)

            @pl.loop(0, n_phases)
            def _phase(p):
                lo = p * R

                @pl.when(s == 0)
                def _stage():
                    pltpu.sync_copy(e_hbm.at[pl.ds(lo, R)], shard_v)

                @pl.loop(0, n_entries // LANES, unroll=8)
                def _rewrite(i):
                    raw = idx_v[pl.ds(i * LANES, LANES)]
                    inr = (raw >= lo) & (raw < lo + R)
                    phase_v[pl.ds(i * LANES, LANES)] = jnp.where(
                        inr, raw - lo, SENT)

                plsc.subcore_barrier()

                @pl.loop(0, n_sub)
                def _chunk(c):
                    dst = acc_v.at[pl.ds(c * CHUNK, CHUNK)]
                    cps = []
                    for k in range(K):
                        src = shard_v.at[plsc.Indices(
                            phase_v.at[pl.ds(k * CB + c * CHUNK, CHUNK)],
                            ignored_value=SENT)]
                        cps.append(pltpu.async_copy(src, dst, sem,
                                                    add=True))
                    for cp in cps:
                        cp.wait()

                plsc.subcore_barrier()

            pltpu.sync_copy(
                acc_v, out_hbm.at[pl.ds(pi * half + w * CB, CB)])

    run = pl.kernel(
        body,
        out_type=jax.ShapeDtypeStruct((B, D), jnp.float32),
        mesh=mesh,
        scratch_types=[
            pltpu.VMEM((n_entries,), jnp.int32),
            pltpu.VMEM((n_entries,), jnp.int32),
            pltpu.VMEM((CB, D), jnp.float32),
            pltpu.VMEM_SHARED((R, D), jnp.float32),
            pltpu.SemaphoreType.DMA,
        ],
    )
    return run(idx3, E, zeros_blk)


def _epilogue(S, genes, doses, types2, Wt, b2, B):
    BLK = 2048
    grid = (B // BLK,)

    def body(s_ref, g_ref, d_ref, t_ref, wt_ref, b_ref, o_ref):
        g = g_ref[...]
        nv = jnp.sum((g >= 0).astype(jnp.float32), axis=1, keepdims=True)
        pooled = s_ref[...] / jnp.maximum(nv, 1.0)
        wt = wt_ref[...]
        mod = (d_ref[...] * wt[0:1, :]
               + jnp.where(t_ref[...] == 0, wt[1:2, :], wt[2:3, :])
               + b_ref[...])
        o_ref[...] = pooled + mod

    return pl.pallas_call(
        body,
        grid=grid,
        in_specs=[
            pl.BlockSpec((BLK, D), lambda i: (i, 0)),
            pl.BlockSpec((BLK, K), lambda i: (i, 0)),
            pl.BlockSpec((BLK, 1), lambda i: (i, 0)),
            pl.BlockSpec((BLK, 1), lambda i: (i, 0)),
            pl.BlockSpec((3, D), lambda i: (0, 0)),
            pl.BlockSpec((1, D), lambda i: (0, 0)),
        ],
        out_specs=pl.BlockSpec((BLK, D), lambda i: (i, 0)),
        out_shape=jax.ShapeDtypeStruct((B, D), jnp.float32),
    )(S, genes, doses, types2, Wt, b2)


def kernel(perturbation_genes, doses, types, E, W, b):
    B = perturbation_genes.shape[0]
    CB = B // (NW * PASSES)

    raw = perturbation_genes.astype(jnp.int32)
    idx3 = (raw.T.reshape(K, PASSES, NW, CB)
            .transpose(1, 2, 0, 3).reshape(PASSES, NW, K * CB))
    zeros_blk = jnp.zeros((CB, D), jnp.float32)

    S = _sc_gather_sum(idx3, E, zeros_blk, B, CB)

    types2 = types.reshape(B, 1).astype(jnp.int32)
    Wt = W.T
    b2 = b.reshape(1, D)
    return _epilogue(S, perturbation_genes, doses, types2, Wt, b2, B)


# speedup vs baseline: 5.7907x; 1.0503x over previous
import functools

import jax
import jax.numpy as jnp
from jax import lax
from jax.experimental import pallas as pl
from jax.experimental.pallas import tpu as pltpu
from jax.experimental.pallas import tpu_sc as plsc

D = 128
K = 20
NC = 2
NS = 16
NW = NC * NS
LANES = 16
CHUNK = 128
R = 10000
SENT = -1
PASSES = 2


def _sc_gather_sum(idx3, E, zeros_blk, B, CB):
    vocab = E.shape[0]
    n_phases = vocab // R
    n_sub = CB // CHUNK
    n_entries = K * CB
    half = B // PASSES
    mesh = plsc.VectorSubcoreMesh(core_axis_name="c", subcore_axis_name="s")

    def body(idx_hbm, e_hbm, z_hbm, out_hbm, idx_v, phase_v, acc_v,
             shard_v, sem):
        w = lax.axis_index("s") * NC + lax.axis_index("c")
        s = lax.axis_index("s")

        n_stage = 10
        r_sub = R // n_stage

        def rewrite(lo, buf):
            @pl.loop(0, n_entries // LANES, unroll=8)
            def _rewrite(i):
                raw = idx_v[pl.ds(i * LANES, LANES)]
                inr = (raw >= lo) & (raw < lo + R)
                phase_v[buf, pl.ds(i * LANES, LANES)] = jnp.where(
                    inr, raw - lo, SENT)

        for pi in range(PASSES):
            pltpu.sync_copy(idx_hbm.at[pi, w], idx_v)
            pltpu.sync_copy(z_hbm, acc_v)
            rewrite(0, 0)

            @pl.loop(0, n_phases)
            def _phase(p):
                lo = p * R
                buf = p % 2

                @pl.when(s < n_stage)
                def _stage():
                    pltpu.sync_copy(
                        e_hbm.at[pl.ds(lo + s * r_sub, r_sub)],
                        shard_v.at[pl.ds(s * r_sub, r_sub)])

                plsc.subcore_barrier()

                cps = []
                for c in range(n_sub):
                    dst = acc_v.at[pl.ds(c * CHUNK, CHUNK)]
                    for k in range(K):
                        src = shard_v.at[plsc.Indices(
                            phase_v.at[buf,
                                       pl.ds(k * CB + c * CHUNK, CHUNK)],
                            ignored_value=SENT)]
                        cps.append(pltpu.async_copy(src, dst, sem,
                                                    add=True))

                @pl.when(p + 1 < n_phases)
                def _next():
                    rewrite((p + 1) * R, 1 - buf)

                for cp in cps:
                    cp.wait()

                plsc.subcore_barrier()

            pltpu.sync_copy(
                acc_v, out_hbm.at[pl.ds(pi * half + w * CB, CB)])

    run = pl.kernel(
        body,
        out_type=jax.ShapeDtypeStruct((B, D), jnp.float32),
        mesh=mesh,
        scratch_types=[
            pltpu.VMEM((n_entries,), jnp.int32),
            pltpu.VMEM((2, n_entries), jnp.int32),
            pltpu.VMEM((CB, D), jnp.float32),
            pltpu.VMEM_SHARED((R, D), jnp.float32),
            pltpu.SemaphoreType.DMA,
        ],
    )
    return run(idx3, E, zeros_blk)


def _epilogue(S, genes, doses, types2, Wt, b2, B):
    BLK = 2048
    grid = (B // BLK,)

    def body(s_ref, g_ref, d_ref, t_ref, wt_ref, b_ref, o_ref):
        g = g_ref[...]
        nv = jnp.sum((g >= 0).astype(jnp.float32), axis=1, keepdims=True)
        pooled = s_ref[...] / jnp.maximum(nv, 1.0)
        wt = wt_ref[...]
        mod = (d_ref[...] * wt[0:1, :]
               + jnp.where(t_ref[...] == 0, wt[1:2, :], wt[2:3, :])
               + b_ref[...])
        o_ref[...] = pooled + mod

    return pl.pallas_call(
        body,
        grid=grid,
        in_specs=[
            pl.BlockSpec((BLK, D), lambda i: (i, 0)),
            pl.BlockSpec((BLK, K), lambda i: (i, 0)),
            pl.BlockSpec((BLK, 1), lambda i: (i, 0)),
            pl.BlockSpec((BLK, 1), lambda i: (i, 0)),
            pl.BlockSpec((3, D), lambda i: (0, 0)),
            pl.BlockSpec((1, D), lambda i: (0, 0)),
        ],
        out_specs=pl.BlockSpec((BLK, D), lambda i: (i, 0)),
        out_shape=jax.ShapeDtypeStruct((B, D), jnp.float32),
    )(S, genes, doses, types2, Wt, b2)


def kernel(perturbation_genes, doses, types, E, W, b):
    B = perturbation_genes.shape[0]
    CB = B // (NW * PASSES)

    raw = perturbation_genes.astype(jnp.int32)
    idx3 = (raw.T.reshape(K, PASSES, NW, CB)
            .transpose(1, 2, 0, 3).reshape(PASSES, NW, K * CB))
    zeros_blk = jnp.zeros((CB, D), jnp.float32)

    S = _sc_gather_sum(idx3, E, zeros_blk, B, CB)

    types2 = types.reshape(B, 1).astype(jnp.int32)
    Wt = W.T
    b2 = b.reshape(1, D)
    return _epilogue(S, perturbation_genes, doses, types2, Wt, b2, B)
